# baseline (speedup 1.0000x reference)
import jax
import jax.numpy as jnp
from jax.experimental import pallas as pl
from jax.experimental.pallas import tpu as pltpu

_N_ITEMS = 1_000_000
_D = 64
_BLOCK = 8_000
_GRID = _N_ITEMS // _BLOCK
_SBLOCK = 40_000
_SGRID = _N_ITEMS // _SBLOCK
_K = 100


def _score_body(uid_ref, user_ref, item_ref, out_ref):
    item_r = item_ref[...].astype(jnp.bfloat16).astype(jnp.float32)
    s = jax.lax.dot_general(
        user_ref[0, :, :], item_r,
        dimension_numbers=(((1,), (1,)), ((), ())),
        preferred_element_type=jnp.float32,
    )
    out_ref[0, :, :] = s


def _scores(user_id, user_emb, item_emb):
    uid = jnp.asarray(user_id, dtype=jnp.int32).reshape((1,))
    user3 = (user_emb.astype(jnp.bfloat16).astype(jnp.float32)
             .reshape((user_emb.shape[0], 1, _D)))
    grid_spec = pltpu.PrefetchScalarGridSpec(
        num_scalar_prefetch=1,
        grid=(_SGRID,),
        in_specs=[
            pl.BlockSpec((1, 1, _D), lambda i, uid_ref: (uid_ref[0], 0, 0)),
            pl.BlockSpec((_SBLOCK, _D), lambda i, uid_ref: (i, 0)),
        ],
        out_specs=pl.BlockSpec((1, 1, _SBLOCK), lambda i, uid_ref: (i, 0, 0)),
    )
    return pl.pallas_call(
        _score_body,
        grid_spec=grid_spec,
        out_shape=jax.ShapeDtypeStruct((_SGRID, 1, _SBLOCK), jnp.float32),
    )(uid, user3, item_emb)


def _topk_body(s_ref, out_ref):
    neg_inf = jnp.float32(-jnp.inf)
    big = jnp.int32(2**30)
    iota_seg = jax.lax.broadcasted_iota(jnp.int32, (_GRID, 1), 0)
    iota_lane = jax.lax.broadcasted_iota(jnp.int32, (1, _BLOCK), 1)
    iota_out = jax.lax.broadcasted_iota(jnp.int32, (1, 128), 1)

    rm = jnp.max(s_ref[...], axis=2)

    def body(t, carry):
        rm, out_row = carry
        m = jnp.max(rm)
        seg = jnp.min(jnp.where(rm == m, iota_seg, big))
        row = s_ref[pl.ds(seg, 1), 0, :]
        lane = jnp.min(jnp.where(row == m, iota_lane, big))
        idx = seg * _BLOCK + lane
        newrow = jnp.where(iota_lane == lane, neg_inf, row)
        s_ref[pl.ds(seg, 1), 0, :] = newrow
        rm = jnp.where(iota_seg == seg, jnp.max(newrow), rm)
        out_row = jnp.where(iota_out == t, idx, out_row)
        return rm, out_row

    _, out_row = jax.lax.fori_loop(
        0, _K, body, (rm, jnp.zeros((1, 128), jnp.int32)))
    out_ref[...] = out_row


def _topk100(scores):
    out = pl.pallas_call(
        _topk_body,
        in_specs=[pl.BlockSpec((_GRID, 1, _BLOCK), lambda: (0, 0, 0))],
        out_specs=pl.BlockSpec((1, 128), lambda: (0, 0)),
        out_shape=jax.ShapeDtypeStruct((1, 128), jnp.int32),
    )(scores)
    return out[0, :_K]


def kernel(user_id, user_emb, item_emb, topk):
    scores = _scores(user_id, user_emb, item_emb)
    return _topk100(scores.reshape((_GRID, 1, _BLOCK)))


# speedup vs baseline: 1.0577x; 1.0577x over previous
import jax
import jax.numpy as jnp
from jax.experimental import pallas as pl
from jax.experimental.pallas import tpu as pltpu

_N_ITEMS = 1_000_000
_D = 64
_BLOCK = 8_000
_GRID = _N_ITEMS // _BLOCK
_STREAMS = 5
_SGRID = _GRID // _STREAMS
_K = 100


def _score_body(uid_ref, user_ref, *refs):
    item_refs = refs[:_STREAMS]
    out_refs = refs[_STREAMS:]
    u = user_ref[0, :, :]
    for s in range(_STREAMS):
        item_r = item_refs[s][...].astype(jnp.bfloat16).astype(jnp.float32)
        sc = jax.lax.dot_general(
            u, item_r,
            dimension_numbers=(((1,), (1,)), ((), ())),
            preferred_element_type=jnp.float32,
        )
        out_refs[s][0, :, :] = sc


def _scores(user_id, user_emb, item_emb):
    uid = jnp.asarray(user_id, dtype=jnp.int32).reshape((1,))
    user3 = (user_emb.astype(jnp.bfloat16).astype(jnp.float32)
             .reshape((user_emb.shape[0], 1, _D)))

    def _item_spec(s):
        return pl.BlockSpec(
            (_BLOCK, _D), lambda i, uid_ref, s=s: (s * _SGRID + i, 0))

    grid_spec = pltpu.PrefetchScalarGridSpec(
        num_scalar_prefetch=1,
        grid=(_SGRID,),
        in_specs=[pl.BlockSpec((1, 1, _D), lambda i, uid_ref: (uid_ref[0], 0, 0))]
        + [_item_spec(s) for s in range(_STREAMS)],
        out_specs=[pl.BlockSpec((1, 1, _BLOCK), lambda i, uid_ref: (i, 0, 0))
                   for _ in range(_STREAMS)],
    )
    outs = pl.pallas_call(
        _score_body,
        grid_spec=grid_spec,
        out_shape=[jax.ShapeDtypeStruct((_SGRID, 1, _BLOCK), jnp.float32)
                   for _ in range(_STREAMS)],
    )(uid, user3, *([item_emb] * _STREAMS))
    return jnp.concatenate(outs, axis=0)


def _topk_body(s_ref, out_ref):
    neg_inf = jnp.float32(-jnp.inf)
    big = jnp.int32(2**30)
    iota_seg = jax.lax.broadcasted_iota(jnp.int32, (_GRID, 1), 0)
    iota_lane = jax.lax.broadcasted_iota(jnp.int32, (1, _BLOCK), 1)
    iota_out = jax.lax.broadcasted_iota(jnp.int32, (1, 128), 1)

    rm = jnp.max(s_ref[...], axis=2)

    def body(t, carry):
        rm, out_row = carry
        m = jnp.max(rm)
        seg = jnp.min(jnp.where(rm == m, iota_seg, big))
        row = s_ref[pl.ds(seg, 1), 0, :]
        lane = jnp.min(jnp.where(row == m, iota_lane, big))
        idx = seg * _BLOCK + lane
        newrow = jnp.where(iota_lane == lane, neg_inf, row)
        s_ref[pl.ds(seg, 1), 0, :] = newrow
        rm = jnp.where(iota_seg == seg, jnp.max(newrow), rm)
        out_row = jnp.where(iota_out == t, idx, out_row)
        return rm, out_row

    _, out_row = jax.lax.fori_loop(
        0, _K, body, (rm, jnp.zeros((1, 128), jnp.int32)))
    out_ref[...] = out_row


def _topk100(scores):
    out = pl.pallas_call(
        _topk_body,
        in_specs=[pl.BlockSpec((_GRID, 1, _BLOCK), lambda: (0, 0, 0))],
        out_specs=pl.BlockSpec((1, 128), lambda: (0, 0)),
        out_shape=jax.ShapeDtypeStruct((1, 128), jnp.int32),
    )(scores)
    return out[0, :_K]


def kernel(user_id, user_emb, item_emb, topk):
    scores = _scores(user_id, user_emb, item_emb)
    return _topk100(scores)


# speedup vs baseline: 1.0709x; 1.0124x over previous
import jax
import jax.numpy as jnp
from jax.experimental import pallas as pl
from jax.experimental.pallas import tpu as pltpu

_N_ITEMS = 1_000_000
_D = 64
_BLOCK = 8_000
_GRID = _N_ITEMS // _BLOCK
_NBUF = 8
_K = 100


def _score_body(uid_ref, user_ref, item_hbm, out_ref, buf, sem):
    u = user_ref[0, :, :]

    def _copy(i, b):
        return pltpu.make_async_copy(
            item_hbm.at[pl.ds(i * _BLOCK, _BLOCK), :],
            buf.at[pl.ds(b * _BLOCK, _BLOCK), :],
            sem.at[b])

    for b in range(_NBUF):
        _copy(b, b).start()

    def step(i, carry):
        b = jax.lax.rem(i, _NBUF)
        _copy(i, b).wait()
        raw = buf[pl.ds(b * _BLOCK, _BLOCK), :]
        item_r = raw.astype(jnp.bfloat16).astype(jnp.float32)
        sc = jax.lax.dot_general(
            u, item_r,
            dimension_numbers=(((1,), (1,)), ((), ())),
            preferred_element_type=jnp.float32,
        )
        out_ref[pl.ds(i, 1), :, :] = sc.reshape(1, 1, _BLOCK)

        @pl.when(i + _NBUF < _GRID)
        def _():
            _copy(i + _NBUF, b).start()
        return carry

    jax.lax.fori_loop(0, _GRID, step, 0)


def _scores(user_id, user_emb, item_emb):
    uid = jnp.asarray(user_id, dtype=jnp.int32).reshape((1,))
    user3 = (user_emb.astype(jnp.bfloat16).astype(jnp.float32)
             .reshape((user_emb.shape[0], 1, _D)))
    grid_spec = pltpu.PrefetchScalarGridSpec(
        num_scalar_prefetch=1,
        grid=(1,),
        in_specs=[
            pl.BlockSpec((1, 1, _D), lambda i, uid_ref: (uid_ref[0], 0, 0)),
            pl.BlockSpec(memory_space=pltpu.MemorySpace.HBM),
        ],
        out_specs=pl.BlockSpec((_GRID, 1, _BLOCK), lambda i, uid_ref: (0, 0, 0)),
        scratch_shapes=[
            pltpu.MemorySpace.VMEM((_NBUF * _BLOCK, _D), jnp.float32),
            pltpu.SemaphoreType.DMA((_NBUF,)),
        ],
    )
    return pl.pallas_call(
        _score_body,
        grid_spec=grid_spec,
        out_shape=jax.ShapeDtypeStruct((_GRID, 1, _BLOCK), jnp.float32),
    )(uid, user3, item_emb)


def _topk_body(s_ref, out_ref):
    neg_inf = jnp.float32(-jnp.inf)
    big = jnp.int32(2**30)
    iota_seg = jax.lax.broadcasted_iota(jnp.int32, (_GRID, 1), 0)
    iota_lane = jax.lax.broadcasted_iota(jnp.int32, (1, _BLOCK), 1)
    iota_out = jax.lax.broadcasted_iota(jnp.int32, (1, 128), 1)

    rm = jnp.max(s_ref[...], axis=2)

    def body(t, carry):
        rm, out_row = carry
        m = jnp.max(rm)
        seg = jnp.min(jnp.where(rm == m, iota_seg, big))
        row = s_ref[pl.ds(seg, 1), 0, :]
        lane = jnp.min(jnp.where(row == m, iota_lane, big))
        idx = seg * _BLOCK + lane
        newrow = jnp.where(iota_lane == lane, neg_inf, row)
        s_ref[pl.ds(seg, 1), 0, :] = newrow
        rm = jnp.where(iota_seg == seg, jnp.max(newrow), rm)
        out_row = jnp.where(iota_out == t, idx, out_row)
        return rm, out_row

    _, out_row = jax.lax.fori_loop(
        0, _K, body, (rm, jnp.zeros((1, 128), jnp.int32)))
    out_ref[...] = out_row


def _topk100(scores):
    out = pl.pallas_call(
        _topk_body,
        in_specs=[pl.BlockSpec((_GRID, 1, _BLOCK), lambda: (0, 0, 0))],
        out_specs=pl.BlockSpec((1, 128), lambda: (0, 0)),
        out_shape=jax.ShapeDtypeStruct((1, 128), jnp.int32),
    )(scores)
    return out[0, :_K]


def kernel(user_id, user_emb, item_emb, topk):
    scores = _scores(user_id, user_emb, item_emb)
    return _topk100(scores)


# speedup vs baseline: 1.1268x; 1.0522x over previous
import jax
import jax.numpy as jnp
from jax.experimental import pallas as pl
from jax.experimental.pallas import tpu as pltpu

_N_ITEMS = 1_000_000
_D = 64
_BLOCK = 8_000
_GRID = _N_ITEMS // _BLOCK
_NBUF = 8
_K = 100


def _fused_body(uid_ref, user_ref, item_hbm, out_ref, sbuf, buf, sem):
    u = user_ref[0, :, :]
    neg_inf = jnp.float32(-jnp.inf)
    big = jnp.int32(2**30)
    iota_seg = jax.lax.broadcasted_iota(jnp.int32, (_GRID, 1), 0)
    iota_lane = jax.lax.broadcasted_iota(jnp.int32, (1, _BLOCK), 1)
    iota_out = jax.lax.broadcasted_iota(jnp.int32, (1, 128), 1)

    def _copy(i, b):
        return pltpu.make_async_copy(
            item_hbm.at[pl.ds(i * _BLOCK, _BLOCK), :],
            buf.at[pl.ds(b * _BLOCK, _BLOCK), :],
            sem.at[b])

    for b in range(_NBUF):
        _copy(b, b).start()

    def step(i, rm):
        b = jax.lax.rem(i, _NBUF)
        _copy(i, b).wait()
        raw = buf[pl.ds(b * _BLOCK, _BLOCK), :]
        item_r = raw.astype(jnp.bfloat16).astype(jnp.float32)
        sc = jax.lax.dot_general(
            u, item_r,
            dimension_numbers=(((1,), (1,)), ((), ())),
            preferred_element_type=jnp.float32,
        )
        sbuf[pl.ds(i, 1), :] = sc
        rm = jnp.where(iota_seg == i, jnp.max(sc), rm)

        @pl.when(i + _NBUF < _GRID)
        def _():
            _copy(i + _NBUF, b).start()
        return rm

    rm0 = jax.lax.fori_loop(
        0, _GRID, step, jnp.full((_GRID, 1), neg_inf, jnp.float32))

    def body(t, carry):
        rm, out_row = carry
        m = jnp.max(rm)
        seg = jnp.min(jnp.where(rm == m, iota_seg, big))
        row = sbuf[pl.ds(seg, 1), :]
        lane = jnp.min(jnp.where(row == m, iota_lane, big))
        idx = seg * _BLOCK + lane
        newrow = jnp.where(iota_lane == lane, neg_inf, row)
        sbuf[pl.ds(seg, 1), :] = newrow
        rm = jnp.where(iota_seg == seg, jnp.max(newrow), rm)
        out_row = jnp.where(iota_out == t, idx, out_row)
        return rm, out_row

    _, out_row = jax.lax.fori_loop(
        0, _K, body, (rm0, jnp.zeros((1, 128), jnp.int32)))
    out_ref[...] = out_row


def kernel(user_id, user_emb, item_emb, topk):
    uid = jnp.asarray(user_id, dtype=jnp.int32).reshape((1,))
    user3 = (user_emb.astype(jnp.bfloat16).astype(jnp.float32)
             .reshape((user_emb.shape[0], 1, _D)))
    grid_spec = pltpu.PrefetchScalarGridSpec(
        num_scalar_prefetch=1,
        grid=(1,),
        in_specs=[
            pl.BlockSpec((1, 1, _D), lambda i, uid_ref: (uid_ref[0], 0, 0)),
            pl.BlockSpec(memory_space=pltpu.MemorySpace.HBM),
        ],
        out_specs=pl.BlockSpec((1, 128), lambda i, uid_ref: (0, 0)),
        scratch_shapes=[
            pltpu.MemorySpace.VMEM((_GRID, _BLOCK), jnp.float32),
            pltpu.MemorySpace.VMEM((_NBUF * _BLOCK, _D), jnp.float32),
            pltpu.SemaphoreType.DMA((_NBUF,)),
        ],
    )
    out = pl.pallas_call(
        _fused_body,
        grid_spec=grid_spec,
        out_shape=jax.ShapeDtypeStruct((1, 128), jnp.int32),
    )(uid, user3, item_emb)
    return out[0, :_K]
